# Initial kernel scaffold; baseline (speedup 1.0000x reference)
import jax
import jax.numpy as jnp
from jax.experimental import pallas as pl


def kernel(node_features, edge_features, edge_index, Wn, bn, W1, b1, W2, b2, W3, b3, W4, b4, Wc, bc):
    raise NotImplementedError("write your pallas kernel here")


# speedup vs baseline: 4.3040x; 4.3040x over previous
import functools

import jax
import jax.numpy as jnp
from jax import lax
from jax.experimental import pallas as pl
from jax.experimental.pallas import tpu as pltpu
from jax.experimental.pallas import tpu_sc as plsc

_N = 10000
_E = 320000
_NPAD = 10240
_EPAD = 327680
_NW = 32
_CHUNK = 128
_EPW = _EPAD // _NW
_CPW = _EPW // _CHUNK
_RPS = _NPAD // 16

_f32 = jnp.float32
_i32 = jnp.int32


def _mesh():
    return plsc.VectorSubcoreMesh(core_axis_name="c", subcore_axis_name="s")


@functools.partial(
    pl.kernel,
    out_type=(jax.ShapeDtypeStruct((2, 2, _NPAD, 16), _f32),
              jax.ShapeDtypeStruct((2, 2, _NPAD, 16), _f32)),
    mesh=_mesh(),
    compiler_params=pltpu.CompilerParams(needs_layout_passes=False,
                                         use_tc_tiling_on_sc=False),
    scratch_types=[
        pltpu.VMEM((_CHUNK,), _i32),
        pltpu.VMEM((_CHUNK,), _i32),
        pltpu.VMEM((_CHUNK, 16), _f32),
        pltpu.VMEM((_CHUNK, 16), _f32),
        pltpu.VMEM_SHARED((_NPAD, 16), _f32),
        pltpu.VMEM_SHARED((_NPAD, 16), _f32),
        pltpu.VMEM_SHARED((_NPAD, 16), _f32),
        pltpu.VMEM_SHARED((_NPAD, 16), _f32),
    ],
)
def _deg_pass(src_hbm, dst_hbm, wexp_hbm, out_s, out_d,
              srcv, dstv, wexp_v, pay, acc_ws, acc_wd, acc_cs, acc_cd):
    c = lax.axis_index("c")
    s = lax.axis_index("s")
    wid = c * 16 + s

    zero16 = jnp.zeros((16,), _f32)

    def _zrow(i, _):
        pay[i, :] = zero16
        return 0

    lax.fori_loop(0, _CHUNK, _zrow, 0)
    for z in range(_RPS // _CHUNK):
        off = pl.ds(s * _RPS + z * _CHUNK, _CHUNK)
        pltpu.sync_copy(pay, acc_ws.at[off])
        pltpu.sync_copy(pay, acc_wd.at[off])
        pltpu.sync_copy(pay, acc_cs.at[off])
        pltpu.sync_copy(pay, acc_cd.at[off])

    ones16 = jnp.ones((16,), _f32)

    def _orow(i, _):
        pay[i, :] = ones16
        return 0

    lax.fori_loop(0, _CHUNK, _orow, 0)
    plsc.subcore_barrier()

    ebase = wid * _EPW

    def _chunk(t, _):
        base = ebase + t * _CHUNK
        pltpu.sync_copy(src_hbm.at[pl.ds(base, _CHUNK)], srcv)
        pltpu.sync_copy(dst_hbm.at[pl.ds(base, _CHUNK)], dstv)
        pltpu.sync_copy(wexp_hbm.at[pl.ds(base, _CHUNK)], wexp_v)
        pltpu.sync_copy(wexp_v, acc_ws.at[srcv], add=True)
        pltpu.sync_copy(wexp_v, acc_wd.at[dstv], add=True)
        pltpu.sync_copy(pay, acc_cs.at[srcv], add=True)
        pltpu.sync_copy(pay, acc_cd.at[dstv], add=True)
        return 0

    lax.fori_loop(0, _CPW, _chunk, 0)
    plsc.subcore_barrier()
    off = pl.ds(s * _RPS, _RPS)
    pltpu.sync_copy(acc_ws.at[off], out_s.at[c, 0, off])
    pltpu.sync_copy(acc_cs.at[off], out_s.at[c, 1, off])
    pltpu.sync_copy(acc_wd.at[off], out_d.at[c, 0, off])
    pltpu.sync_copy(acc_cd.at[off], out_d.at[c, 1, off])


def _make_agg(width, weighted):
    @functools.partial(
        pl.kernel,
        out_type=jax.ShapeDtypeStruct((2, _NPAD, width), _f32),
        mesh=_mesh(),
        compiler_params=pltpu.CompilerParams(
            use_tc_tiling_on_sc=(width % 128 == 0),
            needs_layout_passes=False),
        scratch_types=[
            pltpu.VMEM((_CHUNK,), _i32),
            pltpu.VMEM((_CHUNK,), _i32),
            pltpu.VMEM((_CHUNK, 16), _f32),
            pltpu.VMEM((_CHUNK, width), _f32),
            pltpu.VMEM_SHARED((_NPAD, width), _f32),
            pltpu.SemaphoreType.DMA,
        ],
    )
    def _agg(src_hbm, dst_hbm, w_hbm, table_hbm, out_hbm,
             srcv, dstv, wexp_v, rows, acc, sem):
        c = lax.axis_index("c")
        s = lax.axis_index("s")
        wid = c * 16 + s

        zero16 = jnp.zeros((16,), _f32)

        def _zrow(i, _):
            for j in range(width // 16):
                rows[i, pl.ds(j * 16, 16)] = zero16
            return 0

        lax.fori_loop(0, _CHUNK, _zrow, 0)
        for z in range(_RPS // _CHUNK):
            pltpu.sync_copy(rows, acc.at[pl.ds(s * _RPS + z * _CHUNK, _CHUNK)])
        plsc.subcore_barrier()

        ebase = wid * _EPW

        def _chunk(t, _):
            base = ebase + t * _CHUNK
            pltpu.sync_copy(src_hbm.at[pl.ds(base, _CHUNK)], srcv)
            pltpu.sync_copy(dst_hbm.at[pl.ds(base, _CHUNK)], dstv)
            if weighted:
                pltpu.sync_copy(w_hbm.at[pl.ds(base, _CHUNK)], wexp_v)
            pltpu.async_copy(table_hbm.at[srcv], rows, sem).wait()
            if weighted:
                def _scale(i, _):
                    wsp = wexp_v[i, :]
                    for j in range(width // 16):
                        rows[i, pl.ds(j * 16, 16)] = (
                            rows[i, pl.ds(j * 16, 16)] * wsp)
                    return 0

                lax.fori_loop(0, _CHUNK, _scale, 0)
            pltpu.sync_copy(rows, acc.at[dstv], add=True)
            return 0

        lax.fori_loop(0, _CPW, _chunk, 0)
        plsc.subcore_barrier()
        pltpu.sync_copy(acc.at[pl.ds(s * _RPS, _RPS)],
                        out_hbm.at[c, pl.ds(s * _RPS, _RPS)])

    return _agg


_agg128_w = _make_agg(128, True)
_agg128 = _make_agg(128, False)
_agg64 = _make_agg(64, False)


@functools.partial(
    pl.kernel,
    out_type=jax.ShapeDtypeStruct((_EPAD, 16), _f32),
    mesh=_mesh(),
    compiler_params=pltpu.CompilerParams(use_tc_tiling_on_sc=False,
                                         needs_layout_passes=False),
    scratch_types=[
        pltpu.VMEM((_CHUNK,), _i32),
        pltpu.VMEM((_CHUNK,), _i32),
        pltpu.VMEM((_CHUNK, 16), _f32),
        pltpu.VMEM((_CHUNK, 16), _f32),
        pltpu.SemaphoreType.DMA,
        pltpu.SemaphoreType.DMA,
    ],
)
def _score_pass(src_hbm, dst_hbm, p_hbm, q_hbm, out_hbm,
                srcv, dstv, rp, rq, sem1, sem2):
    c = lax.axis_index("c")
    s = lax.axis_index("s")
    wid = c * 16 + s
    ebase = wid * _EPW

    def _chunk(t, _):
        base = ebase + t * _CHUNK
        pltpu.sync_copy(src_hbm.at[pl.ds(base, _CHUNK)], srcv)
        pltpu.sync_copy(dst_hbm.at[pl.ds(base, _CHUNK)], dstv)
        cp1 = pltpu.async_copy(p_hbm.at[srcv], rp, sem1)
        cp2 = pltpu.async_copy(q_hbm.at[dstv], rq, sem2)
        cp1.wait()
        cp2.wait()

        def _add(i, _):
            rp[i, :] = rp[i, :] + rq[i, :]
            return 0

        lax.fori_loop(0, _CHUNK, _add, 0)
        pltpu.sync_copy(rp, out_hbm.at[pl.ds(base, _CHUNK)])
        return 0

    lax.fori_loop(0, _CPW, _chunk, 0)


_BR = 1024
_GRID = _NPAD // _BR


def _wsum(ref):
    return ref[0, 0, :, 0:1] + ref[1, 0, :, 0:1]


def _csum(ref):
    return ref[0, 1, :, 0:1] + ref[1, 1, :, 0:1]


def _tc1_body(x_ref, wn_ref, bn_ref, ds_ref, out_ref):
    h0 = jnp.dot(x_ref[...], wn_ref[...], preferred_element_type=_f32)
    h0 = h0 + bn_ref[...]
    a = lax.rsqrt(jnp.maximum(_wsum(ds_ref), 1e-12))
    out_ref[...] = h0 * a


def _tc2_body(agg_ref, dd_ref, ds_ref, w1_ref, b1_ref, w2_ref,
              t2a_ref, t2b_ref):
    agg = agg_ref[0] + agg_ref[1]
    b = lax.rsqrt(jnp.maximum(_wsum(dd_ref), 1e-12))
    h1 = jnp.dot(agg * b, w1_ref[...], preferred_element_type=_f32)
    h1 = jnp.maximum(h1 + b1_ref[...], 0.0)
    iso = lax.rsqrt(jnp.maximum(_csum(ds_ref), 1.0))
    t2 = jnp.dot(h1 * iso, w2_ref[...], preferred_element_type=_f32)
    t2a_ref[...] = t2[:, :128]
    t2b_ref[...] = t2[:, 128:]


def _tc3_body(aa_ref, ab_ref, dd_ref, ds_ref, b2_ref, w3_ref, out_ref):
    agg = jnp.concatenate([aa_ref[0] + aa_ref[1], ab_ref[0] + ab_ref[1]],
                          axis=1)
    isi = lax.rsqrt(jnp.maximum(_csum(dd_ref), 1.0))
    h2 = jnp.maximum(agg * isi + b2_ref[...], 0.0)
    iso = lax.rsqrt(jnp.maximum(_csum(ds_ref), 1.0))
    out_ref[...] = jnp.dot(h2 * iso, w3_ref[...], preferred_element_type=_f32)


def _tc4_body(agg_ref, dd_ref, ds_ref, b3_ref, w4_ref, out_ref):
    agg = agg_ref[0] + agg_ref[1]
    isi = lax.rsqrt(jnp.maximum(_csum(dd_ref), 1.0))
    h3 = jnp.maximum(agg * isi + b3_ref[...], 0.0)
    iso = lax.rsqrt(jnp.maximum(_csum(ds_ref), 1.0))
    out_ref[...] = jnp.dot(h3 * iso, w4_ref[...], preferred_element_type=_f32)


def _tc5_body(agg_ref, dd_ref, b4_ref, wct_ref, wcb_ref, bc_ref,
              p_ref, q_ref):
    agg = agg_ref[0] + agg_ref[1]
    isi = lax.rsqrt(jnp.maximum(_csum(dd_ref), 1.0))
    h4 = jnp.maximum(agg * isi + b4_ref[...], 0.0)
    p_ref[...] = jnp.dot(h4, wct_ref[...], preferred_element_type=_f32) + bc_ref[...]
    q_ref[...] = jnp.dot(h4, wcb_ref[...], preferred_element_type=_f32)


def _row_spec(width):
    return pl.BlockSpec((_BR, width), lambda i: (i, 0))


def _deg_spec():
    return pl.BlockSpec((2, 2, _BR, 16), lambda i: (0, 0, i, 0))


def _part_spec(width):
    return pl.BlockSpec((2, _BR, width), lambda i: (0, i, 0))


def _full_spec(shape):
    nd = len(shape)
    return pl.BlockSpec(shape, lambda i: (0,) * nd)


def _tc1(x, wn, bn, deg_s):
    return pl.pallas_call(
        _tc1_body,
        grid=(_GRID,),
        in_specs=[_row_spec(128), _full_spec(wn.shape), _full_spec(bn.shape),
                  _deg_spec()],
        out_specs=_row_spec(128),
        out_shape=jax.ShapeDtypeStruct((_NPAD, 128), _f32),
    )(x, wn, bn, deg_s)


def _tc2(agg1, deg_d, deg_s, w1, b1, w2):
    return pl.pallas_call(
        _tc2_body,
        grid=(_GRID,),
        in_specs=[_part_spec(128), _deg_spec(), _deg_spec(),
                  _full_spec(w1.shape), _full_spec(b1.shape),
                  _full_spec(w2.shape)],
        out_specs=(_row_spec(128), _row_spec(128)),
        out_shape=(jax.ShapeDtypeStruct((_NPAD, 128), _f32),
                   jax.ShapeDtypeStruct((_NPAD, 128), _f32)),
    )(agg1, deg_d, deg_s, w1, b1, w2)


def _tc3(agg2a, agg2b, deg_d, deg_s, b2, w3):
    return pl.pallas_call(
        _tc3_body,
        grid=(_GRID,),
        in_specs=[_part_spec(128), _part_spec(128), _deg_spec(), _deg_spec(),
                  _full_spec(b2.shape), _full_spec(w3.shape)],
        out_specs=_row_spec(128),
        out_shape=jax.ShapeDtypeStruct((_NPAD, 128), _f32),
    )(agg2a, agg2b, deg_d, deg_s, b2, w3)


def _tc4(agg3, deg_d, deg_s, b3, w4):
    return pl.pallas_call(
        _tc4_body,
        grid=(_GRID,),
        in_specs=[_part_spec(128), _deg_spec(), _deg_spec(),
                  _full_spec(b3.shape), _full_spec(w4.shape)],
        out_specs=_row_spec(64),
        out_shape=jax.ShapeDtypeStruct((_NPAD, 64), _f32),
    )(agg3, deg_d, deg_s, b3, w4)


def _tc5(agg4, deg_d, b4, wct, wcb, bc):
    return pl.pallas_call(
        _tc5_body,
        grid=(_GRID,),
        in_specs=[_part_spec(64), _deg_spec(),
                  _full_spec(b4.shape), _full_spec(wct.shape),
                  _full_spec(wcb.shape), _full_spec(bc.shape)],
        out_specs=(_row_spec(16), _row_spec(16)),
        out_shape=(jax.ShapeDtypeStruct((_NPAD, 16), _f32),
                   jax.ShapeDtypeStruct((_NPAD, 16), _f32)),
    )(agg4, deg_d, b4, wct, wcb, bc)


def kernel(node_features, edge_features, edge_index,
           Wn, bn, W1, b1, W2, b2, W3, b3, W4, b4, Wc, bc):
    epad = _EPAD - _E
    src = jnp.concatenate(
        [edge_index[0], jnp.full((epad,), _NPAD - 1, _i32)])
    dst = jnp.concatenate(
        [edge_index[1], jnp.full((epad,), _NPAD - 1, _i32)])
    w = jnp.concatenate([edge_features, jnp.zeros((epad,), _f32)])
    wexp = jnp.broadcast_to(w[:, None], (_EPAD, 16))
    x = jnp.pad(node_features, ((0, _NPAD - _N), (0, 0)))

    deg_s, deg_d = _deg_pass(src, dst, wexp)

    h0p = _tc1(x, Wn, bn.reshape(1, -1), deg_s)
    agg1 = _agg128_w(src, dst, wexp, h0p)

    t2a, t2b = _tc2(agg1, deg_d, deg_s, W1, b1.reshape(1, -1), W2)
    agg2a = _agg128(src, dst, w, t2a)
    agg2b = _agg128(src, dst, w, t2b)

    t3 = _tc3(agg2a, agg2b, deg_d, deg_s, b2.reshape(1, -1), W3)
    agg3 = _agg128(src, dst, w, t3)

    t4 = _tc4(agg3, deg_d, deg_s, b3.reshape(1, -1), W4)
    agg4 = _agg64(src, dst, w, t4)

    p, q = _tc5(agg4, deg_d, b4.reshape(1, -1), Wc[:64], Wc[64:],
                bc.reshape(1, -1))
    scores = _score_pass(src, dst, p, q)
    return scores[:_E]


# speedup vs baseline: 4.9302x; 1.1455x over previous
import functools

import jax
import jax.numpy as jnp
from jax import lax
from jax.experimental import pallas as pl
from jax.experimental.pallas import tpu as pltpu
from jax.experimental.pallas import tpu_sc as plsc

_N = 10000
_E = 320000
_NPAD = 10240
_EPAD = 327680
_NW = 32
_CHUNK = 128
_EPW = _EPAD // _NW
_CPW = _EPW // _CHUNK
_RPS = _NPAD // 16

_f32 = jnp.float32
_i32 = jnp.int32


def _mesh():
    return plsc.VectorSubcoreMesh(core_axis_name="c", subcore_axis_name="s")


@functools.partial(
    pl.kernel,
    out_type=(jax.ShapeDtypeStruct((2, 2, _NPAD, 16), _f32),
              jax.ShapeDtypeStruct((2, 2, _NPAD, 16), _f32)),
    mesh=_mesh(),
    compiler_params=pltpu.CompilerParams(needs_layout_passes=False,
                                         use_tc_tiling_on_sc=False),
    scratch_types=[
        pltpu.VMEM((_CHUNK,), _i32),
        pltpu.VMEM((_CHUNK,), _i32),
        pltpu.VMEM((_CHUNK, 16), _f32),
        pltpu.VMEM((_CHUNK, 16), _f32),
        pltpu.VMEM_SHARED((_NPAD, 16), _f32),
        pltpu.VMEM_SHARED((_NPAD, 16), _f32),
        pltpu.VMEM_SHARED((_NPAD, 16), _f32),
        pltpu.VMEM_SHARED((_NPAD, 16), _f32),
    ],
)
def _deg_pass(src_hbm, dst_hbm, wexp_hbm, out_s, out_d,
              srcv, dstv, wexp_v, pay, acc_ws, acc_wd, acc_cs, acc_cd):
    c = lax.axis_index("c")
    s = lax.axis_index("s")
    wid = c * 16 + s

    zero16 = jnp.zeros((16,), _f32)

    def _zrow(i, _):
        pay[i, :] = zero16
        return 0

    lax.fori_loop(0, _CHUNK, _zrow, 0)
    for z in range(_RPS // _CHUNK):
        off = pl.ds(s * _RPS + z * _CHUNK, _CHUNK)
        pltpu.sync_copy(pay, acc_ws.at[off])
        pltpu.sync_copy(pay, acc_wd.at[off])
        pltpu.sync_copy(pay, acc_cs.at[off])
        pltpu.sync_copy(pay, acc_cd.at[off])

    ones16 = jnp.ones((16,), _f32)

    def _orow(i, _):
        pay[i, :] = ones16
        return 0

    lax.fori_loop(0, _CHUNK, _orow, 0)
    plsc.subcore_barrier()

    ebase = wid * _EPW

    def _chunk(t, _):
        base = ebase + t * _CHUNK
        pltpu.sync_copy(src_hbm.at[pl.ds(base, _CHUNK)], srcv)
        pltpu.sync_copy(dst_hbm.at[pl.ds(base, _CHUNK)], dstv)
        pltpu.sync_copy(wexp_hbm.at[pl.ds(base, _CHUNK)], wexp_v)
        pltpu.sync_copy(wexp_v, acc_ws.at[srcv], add=True)
        pltpu.sync_copy(wexp_v, acc_wd.at[dstv], add=True)
        pltpu.sync_copy(pay, acc_cs.at[srcv], add=True)
        pltpu.sync_copy(pay, acc_cd.at[dstv], add=True)
        return 0

    lax.fori_loop(0, _CPW, _chunk, 0)
    plsc.subcore_barrier()
    off = pl.ds(s * _RPS, _RPS)
    pltpu.sync_copy(acc_ws.at[off], out_s.at[c, 0, off])
    pltpu.sync_copy(acc_cs.at[off], out_s.at[c, 1, off])
    pltpu.sync_copy(acc_wd.at[off], out_d.at[c, 0, off])
    pltpu.sync_copy(acc_cd.at[off], out_d.at[c, 1, off])


def _make_agg(width, weighted):
    ck = 64
    cpw = _EPW // ck

    scratch = [
        pltpu.VMEM((ck,), _i32),
        pltpu.VMEM((ck,), _i32),
        pltpu.VMEM((ck,), _i32),
        pltpu.VMEM((ck,), _i32),
        pltpu.VMEM((ck, width), _f32),
        pltpu.VMEM((ck, width), _f32),
        pltpu.VMEM_SHARED((_NPAD, width), _f32),
        pltpu.SemaphoreType.DMA,
        pltpu.SemaphoreType.DMA,
    ]
    if weighted:
        scratch = ([pltpu.VMEM((ck, 16), _f32), pltpu.VMEM((ck, 16), _f32)]
                   + scratch)

    @functools.partial(
        pl.kernel,
        out_type=jax.ShapeDtypeStruct((2, _NPAD, width), _f32),
        mesh=_mesh(),
        compiler_params=pltpu.CompilerParams(
            use_tc_tiling_on_sc=(width % 128 == 0),
            needs_layout_passes=False),
        scratch_types=scratch,
    )
    def _agg(src_hbm, dst_hbm, w_hbm, table_hbm, out_hbm, *refs):
        if weighted:
            wv0, wv1 = refs[0], refs[1]
            refs = refs[2:]
        srcv0, srcv1, dstv0, dstv1, rows0, rows1, acc, gsem0, gsem1 = refs
        c = lax.axis_index("c")
        s = lax.axis_index("s")
        wid = c * 16 + s

        zero16 = jnp.zeros((16,), _f32)

        def _zrow(i, _):
            for j in range(width // 16):
                rows0[i, pl.ds(j * 16, 16)] = zero16
            return 0

        lax.fori_loop(0, ck, _zrow, 0)
        for z in range(_RPS // ck):
            pltpu.sync_copy(rows0, acc.at[pl.ds(s * _RPS + z * ck, ck)])
        plsc.subcore_barrier()

        ebase = wid * _EPW

        def _scale(rows, wv):
            def body(i, _):
                wsp = wv[i, :]
                for j in range(width // 16):
                    rows[i, pl.ds(j * 16, 16)] = rows[i, pl.ds(j * 16, 16)] * wsp
                return 0
            lax.fori_loop(0, ck, body, 0)

        pltpu.sync_copy(src_hbm.at[pl.ds(ebase, ck)], srcv0)
        pltpu.sync_copy(dst_hbm.at[pl.ds(ebase, ck)], dstv0)
        pltpu.async_copy(table_hbm.at[srcv0], rows0, gsem0)

        def _pair(u, _):
            t0 = u * 2
            b0 = ebase + t0 * ck
            pltpu.sync_copy(src_hbm.at[pl.ds(b0 + ck, ck)], srcv1)
            pltpu.sync_copy(dst_hbm.at[pl.ds(b0 + ck, ck)], dstv1)
            if weighted:
                pltpu.sync_copy(w_hbm.at[pl.ds(b0, ck)], wv0)
            pltpu.make_async_copy(table_hbm.at[srcv0], rows0, gsem0).wait()
            pltpu.async_copy(table_hbm.at[srcv1], rows1, gsem1)
            if weighted:
                _scale(rows0, wv0)
            pltpu.sync_copy(rows0, acc.at[dstv0], add=True)

            @pl.when(u + 1 < cpw // 2)
            def _():
                pltpu.sync_copy(src_hbm.at[pl.ds(b0 + 2 * ck, ck)], srcv0)
                pltpu.sync_copy(dst_hbm.at[pl.ds(b0 + 2 * ck, ck)], dstv0)
                pltpu.async_copy(table_hbm.at[srcv0], rows0, gsem0)

            if weighted:
                pltpu.sync_copy(w_hbm.at[pl.ds(b0 + ck, ck)], wv1)
            pltpu.make_async_copy(table_hbm.at[srcv1], rows1, gsem1).wait()
            if weighted:
                _scale(rows1, wv1)
            pltpu.sync_copy(rows1, acc.at[dstv1], add=True)
            return 0

        lax.fori_loop(0, cpw // 2, _pair, 0)
        plsc.subcore_barrier()
        pltpu.sync_copy(acc.at[pl.ds(s * _RPS, _RPS)],
                        out_hbm.at[c, pl.ds(s * _RPS, _RPS)])

    return _agg


_agg128_w = _make_agg(128, True)
_agg128 = _make_agg(128, False)
_agg64 = _make_agg(64, False)


@functools.partial(
    pl.kernel,
    out_type=jax.ShapeDtypeStruct((_EPAD, 16), _f32),
    mesh=_mesh(),
    compiler_params=pltpu.CompilerParams(use_tc_tiling_on_sc=False,
                                         needs_layout_passes=False),
    scratch_types=[
        pltpu.VMEM((_CHUNK,), _i32),
        pltpu.VMEM((_CHUNK,), _i32),
        pltpu.VMEM((_CHUNK, 16), _f32),
        pltpu.VMEM((_CHUNK, 16), _f32),
        pltpu.SemaphoreType.DMA,
        pltpu.SemaphoreType.DMA,
    ],
)
def _score_pass(src_hbm, dst_hbm, p_hbm, q_hbm, out_hbm,
                srcv, dstv, rp, rq, sem1, sem2):
    c = lax.axis_index("c")
    s = lax.axis_index("s")
    wid = c * 16 + s
    ebase = wid * _EPW

    def _chunk(t, _):
        base = ebase + t * _CHUNK
        pltpu.sync_copy(src_hbm.at[pl.ds(base, _CHUNK)], srcv)
        pltpu.sync_copy(dst_hbm.at[pl.ds(base, _CHUNK)], dstv)
        cp1 = pltpu.async_copy(p_hbm.at[srcv], rp, sem1)
        cp2 = pltpu.async_copy(q_hbm.at[dstv], rq, sem2)
        cp1.wait()
        cp2.wait()

        def _add(i, _):
            rp[i, :] = rp[i, :] + rq[i, :]
            return 0

        lax.fori_loop(0, _CHUNK, _add, 0)
        pltpu.sync_copy(rp, out_hbm.at[pl.ds(base, _CHUNK)])
        return 0

    lax.fori_loop(0, _CPW, _chunk, 0)


_BR = 1024
_GRID = _NPAD // _BR


def _wsum(ref):
    return ref[0, 0, :, 0:1] + ref[1, 0, :, 0:1]


def _csum(ref):
    return ref[0, 1, :, 0:1] + ref[1, 1, :, 0:1]


def _tc1_body(x_ref, wn_ref, bn_ref, ds_ref, out_ref):
    h0 = jnp.dot(x_ref[...], wn_ref[...], preferred_element_type=_f32)
    h0 = h0 + bn_ref[...]
    a = lax.rsqrt(jnp.maximum(_wsum(ds_ref), 1e-12))
    out_ref[...] = h0 * a


def _tc2_body(agg_ref, dd_ref, ds_ref, w1_ref, b1_ref, w2_ref,
              t2a_ref, t2b_ref):
    agg = agg_ref[0] + agg_ref[1]
    b = lax.rsqrt(jnp.maximum(_wsum(dd_ref), 1e-12))
    h1 = jnp.dot(agg * b, w1_ref[...], preferred_element_type=_f32)
    h1 = jnp.maximum(h1 + b1_ref[...], 0.0)
    iso = lax.rsqrt(jnp.maximum(_csum(ds_ref), 1.0))
    t2 = jnp.dot(h1 * iso, w2_ref[...], preferred_element_type=_f32)
    t2a_ref[...] = t2[:, :128]
    t2b_ref[...] = t2[:, 128:]


def _tc3_body(aa_ref, ab_ref, dd_ref, ds_ref, b2_ref, w3_ref, out_ref):
    agg = jnp.concatenate([aa_ref[0] + aa_ref[1], ab_ref[0] + ab_ref[1]],
                          axis=1)
    isi = lax.rsqrt(jnp.maximum(_csum(dd_ref), 1.0))
    h2 = jnp.maximum(agg * isi + b2_ref[...], 0.0)
    iso = lax.rsqrt(jnp.maximum(_csum(ds_ref), 1.0))
    out_ref[...] = jnp.dot(h2 * iso, w3_ref[...], preferred_element_type=_f32)


def _tc4_body(agg_ref, dd_ref, ds_ref, b3_ref, w4_ref, out_ref):
    agg = agg_ref[0] + agg_ref[1]
    isi = lax.rsqrt(jnp.maximum(_csum(dd_ref), 1.0))
    h3 = jnp.maximum(agg * isi + b3_ref[...], 0.0)
    iso = lax.rsqrt(jnp.maximum(_csum(ds_ref), 1.0))
    out_ref[...] = jnp.dot(h3 * iso, w4_ref[...], preferred_element_type=_f32)


def _tc5_body(agg_ref, dd_ref, b4_ref, wct_ref, wcb_ref, bc_ref,
              p_ref, q_ref):
    agg = agg_ref[0] + agg_ref[1]
    isi = lax.rsqrt(jnp.maximum(_csum(dd_ref), 1.0))
    h4 = jnp.maximum(agg * isi + b4_ref[...], 0.0)
    p_ref[...] = jnp.dot(h4, wct_ref[...], preferred_element_type=_f32) + bc_ref[...]
    q_ref[...] = jnp.dot(h4, wcb_ref[...], preferred_element_type=_f32)


def _row_spec(width):
    return pl.BlockSpec((_BR, width), lambda i: (i, 0))


def _deg_spec():
    return pl.BlockSpec((2, 2, _BR, 16), lambda i: (0, 0, i, 0))


def _part_spec(width):
    return pl.BlockSpec((2, _BR, width), lambda i: (0, i, 0))


def _full_spec(shape):
    nd = len(shape)
    return pl.BlockSpec(shape, lambda i: (0,) * nd)


def _tc1(x, wn, bn, deg_s):
    return pl.pallas_call(
        _tc1_body,
        grid=(_GRID,),
        in_specs=[_row_spec(128), _full_spec(wn.shape), _full_spec(bn.shape),
                  _deg_spec()],
        out_specs=_row_spec(128),
        out_shape=jax.ShapeDtypeStruct((_NPAD, 128), _f32),
    )(x, wn, bn, deg_s)


def _tc2(agg1, deg_d, deg_s, w1, b1, w2):
    return pl.pallas_call(
        _tc2_body,
        grid=(_GRID,),
        in_specs=[_part_spec(128), _deg_spec(), _deg_spec(),
                  _full_spec(w1.shape), _full_spec(b1.shape),
                  _full_spec(w2.shape)],
        out_specs=(_row_spec(128), _row_spec(128)),
        out_shape=(jax.ShapeDtypeStruct((_NPAD, 128), _f32),
                   jax.ShapeDtypeStruct((_NPAD, 128), _f32)),
    )(agg1, deg_d, deg_s, w1, b1, w2)


def _tc3(agg2a, agg2b, deg_d, deg_s, b2, w3):
    return pl.pallas_call(
        _tc3_body,
        grid=(_GRID,),
        in_specs=[_part_spec(128), _part_spec(128), _deg_spec(), _deg_spec(),
                  _full_spec(b2.shape), _full_spec(w3.shape)],
        out_specs=_row_spec(128),
        out_shape=jax.ShapeDtypeStruct((_NPAD, 128), _f32),
    )(agg2a, agg2b, deg_d, deg_s, b2, w3)


def _tc4(agg3, deg_d, deg_s, b3, w4):
    return pl.pallas_call(
        _tc4_body,
        grid=(_GRID,),
        in_specs=[_part_spec(128), _deg_spec(), _deg_spec(),
                  _full_spec(b3.shape), _full_spec(w4.shape)],
        out_specs=_row_spec(64),
        out_shape=jax.ShapeDtypeStruct((_NPAD, 64), _f32),
    )(agg3, deg_d, deg_s, b3, w4)


def _tc5(agg4, deg_d, b4, wct, wcb, bc):
    return pl.pallas_call(
        _tc5_body,
        grid=(_GRID,),
        in_specs=[_part_spec(64), _deg_spec(),
                  _full_spec(b4.shape), _full_spec(wct.shape),
                  _full_spec(wcb.shape), _full_spec(bc.shape)],
        out_specs=(_row_spec(16), _row_spec(16)),
        out_shape=(jax.ShapeDtypeStruct((_NPAD, 16), _f32),
                   jax.ShapeDtypeStruct((_NPAD, 16), _f32)),
    )(agg4, deg_d, b4, wct, wcb, bc)


def kernel(node_features, edge_features, edge_index,
           Wn, bn, W1, b1, W2, b2, W3, b3, W4, b4, Wc, bc):
    epad = _EPAD - _E
    src = jnp.concatenate(
        [edge_index[0], jnp.full((epad,), _NPAD - 1, _i32)])
    dst = jnp.concatenate(
        [edge_index[1], jnp.full((epad,), _NPAD - 1, _i32)])
    w = jnp.concatenate([edge_features, jnp.zeros((epad,), _f32)])
    wexp = jnp.broadcast_to(w[:, None], (_EPAD, 16))
    x = jnp.pad(node_features, ((0, _NPAD - _N), (0, 0)))

    deg_s, deg_d = _deg_pass(src, dst, wexp)

    h0p = _tc1(x, Wn, bn.reshape(1, -1), deg_s)
    agg1 = _agg128_w(src, dst, wexp, h0p)

    t2a, t2b = _tc2(agg1, deg_d, deg_s, W1, b1.reshape(1, -1), W2)
    agg2a = _agg128(src, dst, w, t2a)
    agg2b = _agg128(src, dst, w, t2b)

    t3 = _tc3(agg2a, agg2b, deg_d, deg_s, b2.reshape(1, -1), W3)
    agg3 = _agg128(src, dst, w, t3)

    t4 = _tc4(agg3, deg_d, deg_s, b3.reshape(1, -1), W4)
    agg4 = _agg64(src, dst, w, t4)

    p, q = _tc5(agg4, deg_d, b4.reshape(1, -1), Wc[:64], Wc[64:],
                bc.reshape(1, -1))
    scores = _score_pass(src, dst, p, q)
    return scores[:_E]
